# baseline (speedup 1.0000x reference)
import jax
import jax.numpy as jnp
from jax.experimental import pallas as pl
from jax.experimental.pallas import tpu as pltpu

K = 2
TAU = 0.01
MAX_ITER = 100
TOL = 1e-3

N_ROWS = 16384
SUB = 8
LANE = N_ROWS // SUB


def _select_third_largest(c):
    v0 = jnp.minimum(c[0], c[1]); v1 = jnp.maximum(c[0], c[1])
    v2 = jnp.minimum(c[2], c[3]); v3 = jnp.maximum(c[2], c[3])
    v4 = jnp.minimum(c[4], c[5]); v5 = jnp.maximum(c[4], c[5])
    v6 = jnp.minimum(c[6], c[7]); v7 = jnp.maximum(c[6], c[7])
    w2 = jnp.maximum(v0, v2)
    w1 = jnp.minimum(v1, v3); w3 = jnp.maximum(v1, v3)
    w6 = jnp.maximum(v4, v6)
    w5 = jnp.minimum(v5, v7); w7 = jnp.maximum(v5, v7)
    x1 = jnp.minimum(w1, w2); x2 = jnp.maximum(w1, w2)
    x5 = jnp.minimum(w5, w6); x6 = jnp.maximum(w5, w6)
    y5 = jnp.maximum(x1, x5)
    y6 = jnp.maximum(x2, x6)
    y3 = jnp.minimum(w3, w7)
    z5 = jnp.maximum(y3, y5)
    return jnp.minimum(z5, y6)


def _gate_kernel(st_ref, g_ref):
    st = st_ref[...]

    theta0 = _select_third_largest([st[j] for j in range(8)])

    def body(carry):
        theta, i, done = carry
        sig = jax.nn.sigmoid((st - theta[None]) / TAU)
        f = jnp.sum(sig, axis=0) - K
        new_done = (jnp.sum(f) / N_ROWS) < TOL
        df = -(1.0 / TAU) * jnp.sum(sig * (1.0 - sig), axis=0)
        theta_new = theta - f / df
        theta_out = jnp.where(new_done, theta, theta_new)
        return (theta_out, i + 1, new_done)

    def cond(carry):
        _, i, done = carry
        return jnp.logical_and(i < MAX_ITER, jnp.logical_not(done))

    theta, _, _ = jax.lax.while_loop(
        cond, body, (theta0, jnp.int32(0), jnp.bool_(False))
    )

    g_ref[...] = jax.nn.sigmoid((st - theta[None]) / TAU)


@jax.jit
def kernel(s):
    st = s.T.reshape(8, SUB, LANE)
    g_t = pl.pallas_call(
        _gate_kernel,
        out_shape=jax.ShapeDtypeStruct(st.shape, st.dtype),
        in_specs=[pl.BlockSpec(memory_space=pltpu.VMEM)],
        out_specs=pl.BlockSpec(memory_space=pltpu.VMEM),
    )(st)
    return g_t.reshape(8, N_ROWS).T


# speedup vs baseline: 1.9935x; 1.9935x over previous
import jax
import jax.numpy as jnp
from jax.experimental import pallas as pl
from jax.experimental.pallas import tpu as pltpu

K = 2
TAU = 0.01
MAX_ITER = 100
TOL = 1e-3

N_ROWS = 16384
SUB = 8
LANE = N_ROWS // SUB


def _select_third_largest(c):
    v0 = jnp.minimum(c[0], c[1]); v1 = jnp.maximum(c[0], c[1])
    v2 = jnp.minimum(c[2], c[3]); v3 = jnp.maximum(c[2], c[3])
    v4 = jnp.minimum(c[4], c[5]); v5 = jnp.maximum(c[4], c[5])
    v6 = jnp.minimum(c[6], c[7]); v7 = jnp.maximum(c[6], c[7])
    w2 = jnp.maximum(v0, v2)
    w1 = jnp.minimum(v1, v3); w3 = jnp.maximum(v1, v3)
    w6 = jnp.maximum(v4, v6)
    w5 = jnp.minimum(v5, v7); w7 = jnp.maximum(v5, v7)
    x1 = jnp.minimum(w1, w2); x2 = jnp.maximum(w1, w2)
    x5 = jnp.minimum(w5, w6); x6 = jnp.maximum(w5, w6)
    y5 = jnp.maximum(x1, x5)
    y6 = jnp.maximum(x2, x6)
    y3 = jnp.minimum(w3, w7)
    z5 = jnp.maximum(y3, y5)
    return jnp.minimum(z5, y6)


def _gate_kernel(st_ref, g_ref):
    st = st_ref[...].reshape(8, SUB, LANE)

    theta0 = _select_third_largest([st[j] for j in range(8)])

    def body(carry):
        theta, i, done = carry
        sig = jax.nn.sigmoid((st - theta[None]) / TAU)
        f = jnp.sum(sig, axis=0) - K
        new_done = (jnp.sum(f) / N_ROWS) < TOL
        df = -(1.0 / TAU) * jnp.sum(sig * (1.0 - sig), axis=0)
        theta_new = theta - f / df
        theta_out = jnp.where(new_done, theta, theta_new)
        return (theta_out, i + 1, new_done)

    def cond(carry):
        _, i, done = carry
        return jnp.logical_and(i < MAX_ITER, jnp.logical_not(done))

    theta, _, _ = jax.lax.while_loop(
        cond, body, (theta0, jnp.int32(0), jnp.bool_(False))
    )

    g = jax.nn.sigmoid((st - theta[None]) / TAU)
    g_ref[...] = g.reshape(8, N_ROWS)


@jax.jit
def kernel(s):
    st = s.T
    g_t = pl.pallas_call(
        _gate_kernel,
        out_shape=jax.ShapeDtypeStruct(st.shape, st.dtype),
        in_specs=[pl.BlockSpec(memory_space=pltpu.VMEM)],
        out_specs=pl.BlockSpec(memory_space=pltpu.VMEM),
    )(st)
    return g_t.T
